# Initial kernel scaffold; baseline (speedup 1.0000x reference)
import jax
import jax.numpy as jnp
from jax.experimental import pallas as pl


def kernel(h, c, edge_index, node_atts, edges, params):
    raise NotImplementedError("write your pallas kernel here")


# speedup vs baseline: 3.1386x; 3.1386x over previous
import functools

import jax
import jax.numpy as jnp
from jax import lax
from jax.experimental import pallas as pl
from jax.experimental.pallas import tpu as pltpu
from jax.experimental.pallas import tpu_sc as plsc

_NDIM = 128
_GDIM = 128
_HID = 64
_B = 100
_IDX = 100
_N = _B * _IDX
_NA = 8
_ALPHA = 0.5

_NPAD = 10112
_RPT = 632
_K = 128
_EPT = 20480
_EPAD = 16 * _EPT
_E = 320000
_CHUNKS = _EPT // _K

_ROWS_BLK = 1000
_GRID = _N // _ROWS_BLK


def _dot(a, b):
    return jnp.dot(a, b, preferred_element_type=jnp.float32)


def _make_sc_scatter(do_deg):
    mesh = plsc.VectorSubcoreMesh(core_axis_name="c", subcore_axis_name="s")
    out_type = [jax.ShapeDtypeStruct((2 * _NPAD, _NDIM), jnp.float32)]
    scratch = [
        pltpu.VMEM((_K,), jnp.int32),
        pltpu.VMEM((_K,), jnp.int32),
        pltpu.VMEM((_K, _NDIM), jnp.float32),
        pltpu.VMEM_SHARED((_NPAD, _NDIM), jnp.float32),
        pltpu.SemaphoreType.DMA,
    ]
    if do_deg:
        out_type.append(jax.ShapeDtypeStruct((2 * _NPAD, _NDIM), jnp.float32))
        scratch.append(pltpu.VMEM((_K, _NDIM), jnp.float32))

    def body(*refs):
        if do_deg:
            (tab, src_all, dst_all, z128, ones128, s_out, deg_out,
             sidx, didx, rows, acc, sem, onesv) = refs
        else:
            (tab, src_all, dst_all, z128, s_out,
             sidx, didx, rows, acc, sem) = refs
        cc = lax.axis_index("c")
        ss = lax.axis_index("s")
        rbase = ss * _RPT
        obase = cc * _NPAD + rbase
        ebase0 = cc * _EPAD + ss * _EPT
        pltpu.sync_copy(z128, acc.at[pl.ds(rbase, _RPT)])
        if do_deg:
            pltpu.sync_copy(ones128, onesv)
        plsc.subcore_barrier()

        if do_deg:
            def dchunk(j, carry):
                pltpu.sync_copy(dst_all.at[pl.ds(ebase0 + j * _K, _K)], didx)
                pltpu.sync_copy(onesv, acc.at[didx], add=True)
                return carry
            lax.fori_loop(0, _CHUNKS, dchunk, 0)
            plsc.subcore_barrier()
            pltpu.sync_copy(acc.at[pl.ds(rbase, _RPT)],
                            deg_out.at[pl.ds(obase, _RPT)])
            pltpu.sync_copy(z128, acc.at[pl.ds(rbase, _RPT)])
            plsc.subcore_barrier()

        def chunk(j, carry):
            base = ebase0 + j * _K
            pltpu.sync_copy(src_all.at[pl.ds(base, _K)], sidx)
            pltpu.sync_copy(dst_all.at[pl.ds(base, _K)], didx)
            pltpu.async_copy(tab.at[sidx], rows, sem).wait()
            pltpu.sync_copy(rows, acc.at[didx], add=True)
            return carry

        lax.fori_loop(0, _CHUNKS, chunk, 0)
        plsc.subcore_barrier()
        pltpu.sync_copy(acc.at[pl.ds(rbase, _RPT)],
                        s_out.at[pl.ds(obase, _RPT)])

    return pl.kernel(body, mesh=mesh, out_type=out_type, scratch_types=scratch)


def _sc_layer1(tab, src_all, dst_all, z128, ones128):
    res = _make_sc_scatter(True)(tab, src_all, dst_all, z128, ones128)
    return res[0], res[1]


def _sc_layer2(tab, src_all, dst_all, z128):
    res = _make_sc_scatter(False)(tab, src_all, dst_all, z128)
    return res[0] if isinstance(res, (list, tuple)) else res


def _full_spec(arr):
    return pl.BlockSpec(arr.shape, lambda i: tuple(0 for _ in arr.shape))


def _row_spec(ncols):
    return pl.BlockSpec((_ROWS_BLK, ncols), lambda i: (i, 0))


def _gru(x, h, p):
    (wxr, whr, bir, bhr, wxz, whz, biz, bhz, wxn, whn, bin_, bhn) = p
    r = jax.nn.sigmoid(_dot(x, wxr) + bir + _dot(h, whr) + bhr)
    z = jax.nn.sigmoid(_dot(x, wxz) + biz + _dot(h, whz) + bhz)
    n = jnp.tanh(_dot(x, wxn) + bin_ + r * (_dot(h, whn) + bhn))
    return (1.0 - z) * n + z * h


def _pre_body(hf, hb, wlf, wlb, a_out):
    a_out[0] = _dot(hf[...], wlf[...])
    a_out[1] = _dot(hb[...], wlb[...])


def _mid_body(*refs):
    hf, hb, sf, sb, dgf, dgb = refs[:6]
    fw = refs[6:21]
    bw = refs[21:36]
    hf2o, hb2o, af2o = refs[36:39]

    def side(h_ref, s_ref, dg_ref, pr, h2o, a2o, plane):
        h = h_ref[...]
        deg = dg_ref[...][:, 0:1]
        wrT, mb = pr[0][...], pr[1][...]
        aggr = s_ref[...] + deg * (_dot(h, wrT) + mb)
        gru_p = [r[...] for r in pr[2:14]]
        h2 = _gru(aggr, h, gru_p)
        h2o[...] = h2
        a2o[plane] = _dot(h2, pr[14][...])

    side(hf, sf, dgf, fw, hf2o, af2o, 0)
    side(hb, sb, dgb, bw, hb2o, af2o, 1)


def _post_body(*refs):
    hf, hb, sf, sb, dgf, dgb = refs[:6]
    fw = refs[6:20]
    bw = refs[20:34]
    (fm1, fb1, gm1, gb1, fm2, fb2, gm2, gb2, waT) = refs[34:43]
    hno, u1o, u2o, hao = refs[43:47]

    def side(h_ref, s_ref, dg_ref, pr):
        h = h_ref[...]
        deg = dg_ref[...][:, 0:1]
        wrT, mb = pr[0][...], pr[1][...]
        aggr = s_ref[...] + deg * (_dot(h, wrT) + mb)
        gru_p = [r[...] for r in pr[2:14]]
        return _gru(aggr, h, gru_p)

    hf3 = side(hf, sf, dgf, fw)
    hb3 = side(hb, sb, dgb, bw)
    hn = jnp.concatenate([hf3, hb3], axis=1)
    hno[...] = hn
    u1o[...] = (_dot(hn, fm1[...]) + fb1[...]) * jax.nn.sigmoid(
        _dot(hn, gm1[...]) + gb1[...])
    u2o[...] = (_dot(hn, fm2[...]) + fb2[...]) * jax.nn.sigmoid(
        _dot(hn, gm2[...]) + gb2[...])
    hao[...] = _dot(hn, waT[...])


def _tail_body(u1, u2, ha, rm, ttile, trep, c, onehot, edges_flat,
               fanWgT, fanWcT, fanb, fan2T, fan2b, ninits,
               fin_eT, fin_gT, fin_cT, finb, fin2T, fin2b,
               wbT, wcT, wdT, fs1b, fs2T, fs2b,
               loss_o, hv_o):
    R = rm[...]
    hg = _dot(R, u1[...])
    hgi = _dot(R, u2[...])
    cv = c[...]
    s = _dot(hg, fanWgT[...]) + _dot(cv, fanWcT[...]) + fanb[...]
    ns = _dot(jax.nn.relu(s), fan2T[...]) + fan2b[...]
    m = jnp.max(ns, axis=1, keepdims=True)
    logp = ns - m - jnp.log(jnp.sum(jnp.exp(ns - m), axis=1, keepdims=True))
    oh = onehot[...]
    node_loss = -jnp.sum(logp * oh, axis=1, keepdims=True)
    e = _dot(oh, ninits[...])
    t = jax.nn.relu(_dot(e, fin_eT[...]) + _dot(hgi, fin_gT[...]) +
                    _dot(cv, fin_cT[...]) + finb[...])
    hv = _dot(t, fin2T[...]) + fin2b[...]
    hv_o[...] = hv
    P = _dot(hg, wcT[...]) + _dot(cv, wdT[...]) + fs1b[...]
    Q = _dot(hv, wbT[...])
    s2 = ha[...] + _dot(ttile[...], P) + _dot(trep[...], Q)
    es = _dot(jax.nn.relu(s2), fs2T[...]) + fs2b[...]
    ev = edges_flat[...]
    bce = (jnp.maximum(es, 0.0) - es * ev +
           jnp.log(1.0 + jnp.exp(-jnp.abs(es))))
    edge_loss = _dot(R, bce) * (1.0 / _IDX)
    loss_o[...] = 2.0 * ((1.0 - _ALPHA) * node_loss + _ALPHA * edge_loss)


def _prep_layer(p):
    wih, whh = p['Wih'], p['Whh']
    bih, bhh = p['bih'], p['bhh']
    return [
        p['msg_W'][:, _HID:].T,
        p['msg_b'][None, :],
        wih[0:_HID].T, whh[0:_HID].T,
        bih[None, 0:_HID], bhh[None, 0:_HID],
        wih[_HID:2 * _HID].T, whh[_HID:2 * _HID].T,
        bih[None, _HID:2 * _HID], bhh[None, _HID:2 * _HID],
        wih[2 * _HID:].T, whh[2 * _HID:].T,
        bih[None, 2 * _HID:], bhh[None, 2 * _HID:],
    ]


def kernel(h, c, edge_index, node_atts, edges, params):
    f32 = jnp.float32
    h_flat = h.reshape(_N, _NDIM)
    h_f = h_flat[:, :_HID]
    h_b = h_flat[:, _HID:]

    pad_src = jnp.zeros((_EPAD - _E,), jnp.int32)
    pad_dst = jnp.full((_EPAD - _E,), _N, jnp.int32)
    srcf = jnp.concatenate([edge_index[0], pad_src])
    dstf = jnp.concatenate([edge_index[1], pad_dst])
    srcb = jnp.concatenate([edge_index[1], pad_src]) + _N
    dstb = jnp.concatenate([edge_index[0], pad_dst])
    src_all = jnp.concatenate([srcf, srcb])
    dst_all = jnp.concatenate([dstf, dstb])

    z128 = jnp.zeros((_RPT, _NDIM), f32)
    ones128 = jnp.ones((_K, _NDIM), f32)

    fl1, fl2 = params['fwd_layers']
    bl1, bl2 = params['bwd_layers']
    wl_f1 = fl1['msg_W'][:, :_HID].T
    wl_b1 = bl1['msg_W'][:, :_HID].T
    wl_f2 = fl2['msg_W'][:, :_HID].T
    wl_b2 = bl2['msg_W'][:, :_HID].T

    a1 = pl.pallas_call(
        _pre_body,
        grid=(_GRID,),
        in_specs=[_row_spec(_HID), _row_spec(_HID),
                  _full_spec(wl_f1), _full_spec(wl_b1)],
        out_specs=pl.BlockSpec((2, _ROWS_BLK, _NDIM), lambda i: (0, i, 0)),
        out_shape=jax.ShapeDtypeStruct((2, _N, _NDIM), f32),
    )(h_f, h_b, wl_f1, wl_b1)

    s1, deg = _sc_layer1(a1.reshape(2 * _N, _NDIM), src_all, dst_all,
                         z128, ones128)
    s_f1, s_b1 = s1[:_N], s1[_NPAD:_NPAD + _N]
    deg_f, deg_b = deg[:_N], deg[_NPAD:_NPAD + _N]

    mid_params = ([h_f, h_b, s_f1, s_b1, deg_f, deg_b]
                  + _prep_layer(fl1) + [wl_f2]
                  + _prep_layer(bl1) + [wl_b2])
    mid_specs = ([_row_spec(_HID), _row_spec(_HID),
                  _row_spec(_NDIM), _row_spec(_NDIM),
                  _row_spec(_NDIM), _row_spec(_NDIM)]
                 + [_full_spec(a) for a in mid_params[6:]])
    h_f2, h_b2, a2 = pl.pallas_call(
        _mid_body,
        grid=(_GRID,),
        in_specs=mid_specs,
        out_specs=[_row_spec(_HID), _row_spec(_HID),
                   pl.BlockSpec((2, _ROWS_BLK, _NDIM), lambda i: (0, i, 0))],
        out_shape=[jax.ShapeDtypeStruct((_N, _HID), f32),
                   jax.ShapeDtypeStruct((_N, _HID), f32),
                   jax.ShapeDtypeStruct((2, _N, _NDIM), f32)],
    )(*mid_params)

    s2 = _sc_layer2(a2.reshape(2 * _N, _NDIM), src_all, dst_all, z128)
    s_f2, s_b2 = s2[:_N], s2[_NPAD:_NPAD + _N]

    ge, gei = params['graph_emb'], params['graph_emb_init']
    wa_T = params['fs1_W'][:, :_NDIM].T
    post_params = ([h_f2, h_b2, s_f2, s_b2, deg_f, deg_b]
                   + _prep_layer(fl2) + _prep_layer(bl2)
                   + [ge['fm_W'].T, ge['fm_b'][None, :],
                      ge['gm_W'].T, ge['gm_b'][None, :],
                      gei['fm_W'].T, gei['fm_b'][None, :],
                      gei['gm_W'].T, gei['gm_b'][None, :],
                      wa_T])
    post_specs = ([_row_spec(_HID), _row_spec(_HID),
                   _row_spec(_NDIM), _row_spec(_NDIM),
                   _row_spec(_NDIM), _row_spec(_NDIM)]
                  + [_full_spec(a) for a in post_params[6:]])
    hn, u1, u2, ha = pl.pallas_call(
        _post_body,
        grid=(_GRID,),
        in_specs=post_specs,
        out_specs=[_row_spec(_NDIM), _row_spec(_GDIM),
                   _row_spec(_GDIM), _row_spec(2 * _GDIM)],
        out_shape=[jax.ShapeDtypeStruct((_N, _NDIM), f32),
                   jax.ShapeDtypeStruct((_N, _GDIM), f32),
                   jax.ShapeDtypeStruct((_N, _GDIM), f32),
                   jax.ShapeDtypeStruct((_N, 2 * _GDIM), f32)],
    )(*post_params)

    eye = jnp.eye(_IDX, dtype=f32)
    t_tile = jnp.tile(eye, (_B, 1))
    t_rep = jnp.repeat(eye, _IDX, axis=0)
    rm = t_rep.T
    onehot = jax.nn.one_hot(node_atts, _NA, dtype=f32)
    edges_flat = edges.reshape(_N, 1)
    fan_W, fs1_W = params['fan_W'], params['fs1_W']
    tail_in = [u1, u2, ha, rm, t_tile, t_rep, c, onehot, edges_flat,
               fan_W[:, :_GDIM].T, fan_W[:, _GDIM:].T,
               params['fan_b'][None, :],
               params['fan2_W'].T, params['fan2_b'][None, :],
               params['node_inits'],
               params['finit_W'][:, :_NDIM].T,
               params['finit_W'][:, _NDIM:_NDIM + _GDIM].T,
               params['finit_W'][:, _NDIM + _GDIM:].T,
               params['finit_b'][None, :],
               params['finit2_W'].T, params['finit2_b'][None, :],
               fs1_W[:, _NDIM:_NDIM + _GDIM].T,
               fs1_W[:, _NDIM + _GDIM:_NDIM + 2 * _GDIM].T,
               fs1_W[:, _NDIM + 2 * _GDIM:].T,
               params['fs1_b'][None, :],
               params['fs2_W'].T, params['fs2_b'][None, :]]
    loss2d, h_v = pl.pallas_call(
        _tail_body,
        out_shape=[jax.ShapeDtypeStruct((_B, 1), f32),
                   jax.ShapeDtypeStruct((_B, _NDIM), f32)],
    )(*tail_in)

    h_out = jnp.concatenate([hn.reshape(_B, _IDX, _NDIM), h_v[:, None, :]],
                            axis=1)
    return (h_out, loss2d[:, 0])


# speedup vs baseline: 3.5501x; 1.1311x over previous
import functools

import jax
import jax.numpy as jnp
from jax import lax
from jax.experimental import pallas as pl
from jax.experimental.pallas import tpu as pltpu
from jax.experimental.pallas import tpu_sc as plsc

_NDIM = 128
_GDIM = 128
_HID = 64
_B = 100
_IDX = 100
_N = _B * _IDX
_NA = 8
_ALPHA = 0.5

_NPAD = 10112
_RPT = 632
_K = 128
_EPT = 20480
_EPAD = 16 * _EPT
_E = 320000
_CHUNKS = _EPT // _K
_G = 2
_GROUPS = _CHUNKS // _G

_ROWS_BLK = 1000
_GRID = _N // _ROWS_BLK


def _dot(a, b):
    return jnp.dot(a, b, preferred_element_type=jnp.float32)


def _make_sc_scatter(do_deg):
    mesh = plsc.VectorSubcoreMesh(core_axis_name="c", subcore_axis_name="s")
    out_type = [jax.ShapeDtypeStruct((2 * _NPAD, _NDIM), jnp.float32)]
    scratch = [
        pltpu.VMEM((_G, _K), jnp.int32),
        pltpu.VMEM((_G, _K), jnp.int32),
        pltpu.VMEM((_G, _K, _NDIM), jnp.float32),
        pltpu.VMEM_SHARED((_NPAD, _NDIM), jnp.float32),
        pltpu.SemaphoreType.DMA,
        pltpu.SemaphoreType.DMA,
    ]
    if do_deg:
        out_type.append(jax.ShapeDtypeStruct((2 * _NPAD, _NDIM), jnp.float32))

    def body(*refs):
        if do_deg:
            (tab, src2d, dst2d, z128, ones128, s_out, deg_out,
             sidx2, didx2, rows, acc, sem_g, sem_s) = refs
        else:
            (tab, src2d, dst2d, z128, s_out,
             sidx2, didx2, rows, acc, sem_g, sem_s) = refs
        cc = lax.axis_index("c")
        ss = lax.axis_index("s")
        rbase = ss * _RPT
        obase = cc * _NPAD + rbase
        irow0 = cc * (_EPAD // _K) + ss * _CHUNKS
        pltpu.sync_copy(z128, acc.at[pl.ds(rbase, _RPT)])
        if do_deg:
            pltpu.sync_copy(ones128, rows.at[0])
        plsc.subcore_barrier()

        if do_deg:
            def dgroup(g, carry):
                pltpu.sync_copy(dst2d.at[pl.ds(irow0 + g * _G, _G)], didx2)
                ds_ = [pltpu.async_copy(rows.at[0], acc.at[didx2.at[u]],
                                        sem_s, add=True) for u in range(_G)]
                for d in ds_:
                    d.wait()
                return carry
            lax.fori_loop(0, _GROUPS, dgroup, 0)
            plsc.subcore_barrier()
            pltpu.sync_copy(acc.at[pl.ds(rbase, _RPT)],
                            deg_out.at[pl.ds(obase, _RPT)])
            pltpu.sync_copy(z128, acc.at[pl.ds(rbase, _RPT)])
            plsc.subcore_barrier()

        def group(g, carry):
            pltpu.sync_copy(src2d.at[pl.ds(irow0 + g * _G, _G)], sidx2)
            pltpu.sync_copy(dst2d.at[pl.ds(irow0 + g * _G, _G)], didx2)
            dg = [pltpu.async_copy(tab.at[sidx2.at[u]], rows.at[u], sem_g)
                  for u in range(_G)]
            for d in dg:
                d.wait()
            ds_ = [pltpu.async_copy(rows.at[u], acc.at[didx2.at[u]], sem_s,
                                    add=True) for u in range(_G)]
            for d in ds_:
                d.wait()
            return carry

        lax.fori_loop(0, _GROUPS, group, 0)
        plsc.subcore_barrier()
        pltpu.sync_copy(acc.at[pl.ds(rbase, _RPT)],
                        s_out.at[pl.ds(obase, _RPT)])

    return pl.kernel(body, mesh=mesh, out_type=out_type, scratch_types=scratch)


def _sc_layer1(tab, src_all, dst_all, z128, ones128):
    res = _make_sc_scatter(True)(tab, src_all, dst_all, z128, ones128)
    return res[0], res[1]


def _sc_layer2(tab, src_all, dst_all, z128):
    res = _make_sc_scatter(False)(tab, src_all, dst_all, z128)
    return res[0] if isinstance(res, (list, tuple)) else res


def _full_spec(arr):
    return pl.BlockSpec(arr.shape, lambda i: tuple(0 for _ in arr.shape))


def _row_spec(ncols):
    return pl.BlockSpec((_ROWS_BLK, ncols), lambda i: (i, 0))


def _gru(x, h, p):
    (wxr, whr, bir, bhr, wxz, whz, biz, bhz, wxn, whn, bin_, bhn) = p
    r = jax.nn.sigmoid(_dot(x, wxr) + bir + _dot(h, whr) + bhr)
    z = jax.nn.sigmoid(_dot(x, wxz) + biz + _dot(h, whz) + bhz)
    n = jnp.tanh(_dot(x, wxn) + bin_ + r * (_dot(h, whn) + bhn))
    return (1.0 - z) * n + z * h


def _pre_body(hf, hb, wlf, wlb, a_out):
    a_out[0] = _dot(hf[...], wlf[...])
    a_out[1] = _dot(hb[...], wlb[...])


def _mid_body(*refs):
    hf, hb, sf, sb, dgf, dgb = refs[:6]
    fw = refs[6:21]
    bw = refs[21:36]
    hf2o, hb2o, af2o = refs[36:39]

    def side(h_ref, s_ref, dg_ref, pr, h2o, a2o, plane):
        h = h_ref[...]
        deg = dg_ref[...][:, 0:1]
        wrT, mb = pr[0][...], pr[1][...]
        aggr = s_ref[...] + deg * (_dot(h, wrT) + mb)
        gru_p = [r[...] for r in pr[2:14]]
        h2 = _gru(aggr, h, gru_p)
        h2o[...] = h2
        a2o[plane] = _dot(h2, pr[14][...])

    side(hf, sf, dgf, fw, hf2o, af2o, 0)
    side(hb, sb, dgb, bw, hb2o, af2o, 1)


def _post_body(*refs):
    hf, hb, sf, sb, dgf, dgb = refs[:6]
    fw = refs[6:20]
    bw = refs[20:34]
    (fm1, fb1, gm1, gb1, fm2, fb2, gm2, gb2, waT) = refs[34:43]
    hno, u1o, u2o, hao = refs[43:47]

    def side(h_ref, s_ref, dg_ref, pr):
        h = h_ref[...]
        deg = dg_ref[...][:, 0:1]
        wrT, mb = pr[0][...], pr[1][...]
        aggr = s_ref[...] + deg * (_dot(h, wrT) + mb)
        gru_p = [r[...] for r in pr[2:14]]
        return _gru(aggr, h, gru_p)

    hf3 = side(hf, sf, dgf, fw)
    hb3 = side(hb, sb, dgb, bw)
    hn = jnp.concatenate([hf3, hb3], axis=1)
    hno[...] = hn
    u1o[...] = (_dot(hn, fm1[...]) + fb1[...]) * jax.nn.sigmoid(
        _dot(hn, gm1[...]) + gb1[...])
    u2o[...] = (_dot(hn, fm2[...]) + fb2[...]) * jax.nn.sigmoid(
        _dot(hn, gm2[...]) + gb2[...])
    hao[...] = _dot(hn, waT[...])


def _tail_body(u1, u2, ha, rm, ttile, trep, c, onehot, edges_flat,
               fanWgT, fanWcT, fanb, fan2T, fan2b, ninits,
               fin_eT, fin_gT, fin_cT, finb, fin2T, fin2b,
               wbT, wcT, wdT, fs1b, fs2T, fs2b,
               loss_o, hv_o):
    R = rm[...]
    hg = _dot(R, u1[...])
    hgi = _dot(R, u2[...])
    cv = c[...]
    s = _dot(hg, fanWgT[...]) + _dot(cv, fanWcT[...]) + fanb[...]
    ns = _dot(jax.nn.relu(s), fan2T[...]) + fan2b[...]
    m = jnp.max(ns, axis=1, keepdims=True)
    logp = ns - m - jnp.log(jnp.sum(jnp.exp(ns - m), axis=1, keepdims=True))
    oh = onehot[...]
    node_loss = -jnp.sum(logp * oh, axis=1, keepdims=True)
    e = _dot(oh, ninits[...])
    t = jax.nn.relu(_dot(e, fin_eT[...]) + _dot(hgi, fin_gT[...]) +
                    _dot(cv, fin_cT[...]) + finb[...])
    hv = _dot(t, fin2T[...]) + fin2b[...]
    hv_o[...] = hv
    P = _dot(hg, wcT[...]) + _dot(cv, wdT[...]) + fs1b[...]
    Q = _dot(hv, wbT[...])
    s2 = ha[...] + _dot(ttile[...], P) + _dot(trep[...], Q)
    es = _dot(jax.nn.relu(s2), fs2T[...]) + fs2b[...]
    ev = edges_flat[...]
    bce = (jnp.maximum(es, 0.0) - es * ev +
           jnp.log(1.0 + jnp.exp(-jnp.abs(es))))
    edge_loss = _dot(R, bce) * (1.0 / _IDX)
    loss_o[...] = 2.0 * ((1.0 - _ALPHA) * node_loss + _ALPHA * edge_loss)


def _prep_layer(p):
    wih, whh = p['Wih'], p['Whh']
    bih, bhh = p['bih'], p['bhh']
    return [
        p['msg_W'][:, _HID:].T,
        p['msg_b'][None, :],
        wih[0:_HID].T, whh[0:_HID].T,
        bih[None, 0:_HID], bhh[None, 0:_HID],
        wih[_HID:2 * _HID].T, whh[_HID:2 * _HID].T,
        bih[None, _HID:2 * _HID], bhh[None, _HID:2 * _HID],
        wih[2 * _HID:].T, whh[2 * _HID:].T,
        bih[None, 2 * _HID:], bhh[None, 2 * _HID:],
    ]


def kernel(h, c, edge_index, node_atts, edges, params):
    f32 = jnp.float32
    h_flat = h.reshape(_N, _NDIM)
    h_f = h_flat[:, :_HID]
    h_b = h_flat[:, _HID:]

    pad_src = jnp.zeros((_EPAD - _E,), jnp.int32)
    pad_dst = jnp.full((_EPAD - _E,), _N, jnp.int32)
    srcf = jnp.concatenate([edge_index[0], pad_src])
    dstf = jnp.concatenate([edge_index[1], pad_dst])
    srcb = jnp.concatenate([edge_index[1], pad_src]) + _N
    dstb = jnp.concatenate([edge_index[0], pad_dst])
    src_all = jnp.concatenate([srcf, srcb]).reshape(2 * _EPAD // _K, _K)
    dst_all = jnp.concatenate([dstf, dstb]).reshape(2 * _EPAD // _K, _K)

    z128 = jnp.zeros((_RPT, _NDIM), f32)
    ones128 = jnp.ones((_K, _NDIM), f32)

    fl1, fl2 = params['fwd_layers']
    bl1, bl2 = params['bwd_layers']
    wl_f1 = fl1['msg_W'][:, :_HID].T
    wl_b1 = bl1['msg_W'][:, :_HID].T
    wl_f2 = fl2['msg_W'][:, :_HID].T
    wl_b2 = bl2['msg_W'][:, :_HID].T

    a1 = pl.pallas_call(
        _pre_body,
        grid=(_GRID,),
        in_specs=[_row_spec(_HID), _row_spec(_HID),
                  _full_spec(wl_f1), _full_spec(wl_b1)],
        out_specs=pl.BlockSpec((2, _ROWS_BLK, _NDIM), lambda i: (0, i, 0)),
        out_shape=jax.ShapeDtypeStruct((2, _N, _NDIM), f32),
    )(h_f, h_b, wl_f1, wl_b1)

    s1, deg = _sc_layer1(a1.reshape(2 * _N, _NDIM), src_all, dst_all,
                         z128, ones128)
    s_f1, s_b1 = s1[:_N], s1[_NPAD:_NPAD + _N]
    deg_f, deg_b = deg[:_N], deg[_NPAD:_NPAD + _N]

    mid_params = ([h_f, h_b, s_f1, s_b1, deg_f, deg_b]
                  + _prep_layer(fl1) + [wl_f2]
                  + _prep_layer(bl1) + [wl_b2])
    mid_specs = ([_row_spec(_HID), _row_spec(_HID),
                  _row_spec(_NDIM), _row_spec(_NDIM),
                  _row_spec(_NDIM), _row_spec(_NDIM)]
                 + [_full_spec(a) for a in mid_params[6:]])
    h_f2, h_b2, a2 = pl.pallas_call(
        _mid_body,
        grid=(_GRID,),
        in_specs=mid_specs,
        out_specs=[_row_spec(_HID), _row_spec(_HID),
                   pl.BlockSpec((2, _ROWS_BLK, _NDIM), lambda i: (0, i, 0))],
        out_shape=[jax.ShapeDtypeStruct((_N, _HID), f32),
                   jax.ShapeDtypeStruct((_N, _HID), f32),
                   jax.ShapeDtypeStruct((2, _N, _NDIM), f32)],
    )(*mid_params)

    s2 = _sc_layer2(a2.reshape(2 * _N, _NDIM), src_all, dst_all, z128)
    s_f2, s_b2 = s2[:_N], s2[_NPAD:_NPAD + _N]

    ge, gei = params['graph_emb'], params['graph_emb_init']
    wa_T = params['fs1_W'][:, :_NDIM].T
    post_params = ([h_f2, h_b2, s_f2, s_b2, deg_f, deg_b]
                   + _prep_layer(fl2) + _prep_layer(bl2)
                   + [ge['fm_W'].T, ge['fm_b'][None, :],
                      ge['gm_W'].T, ge['gm_b'][None, :],
                      gei['fm_W'].T, gei['fm_b'][None, :],
                      gei['gm_W'].T, gei['gm_b'][None, :],
                      wa_T])
    post_specs = ([_row_spec(_HID), _row_spec(_HID),
                   _row_spec(_NDIM), _row_spec(_NDIM),
                   _row_spec(_NDIM), _row_spec(_NDIM)]
                  + [_full_spec(a) for a in post_params[6:]])
    hn, u1, u2, ha = pl.pallas_call(
        _post_body,
        grid=(_GRID,),
        in_specs=post_specs,
        out_specs=[_row_spec(_NDIM), _row_spec(_GDIM),
                   _row_spec(_GDIM), _row_spec(2 * _GDIM)],
        out_shape=[jax.ShapeDtypeStruct((_N, _NDIM), f32),
                   jax.ShapeDtypeStruct((_N, _GDIM), f32),
                   jax.ShapeDtypeStruct((_N, _GDIM), f32),
                   jax.ShapeDtypeStruct((_N, 2 * _GDIM), f32)],
    )(*post_params)

    eye = jnp.eye(_IDX, dtype=f32)
    t_tile = jnp.tile(eye, (_B, 1))
    t_rep = jnp.repeat(eye, _IDX, axis=0)
    rm = t_rep.T
    onehot = jax.nn.one_hot(node_atts, _NA, dtype=f32)
    edges_flat = edges.reshape(_N, 1)
    fan_W, fs1_W = params['fan_W'], params['fs1_W']
    tail_in = [u1, u2, ha, rm, t_tile, t_rep, c, onehot, edges_flat,
               fan_W[:, :_GDIM].T, fan_W[:, _GDIM:].T,
               params['fan_b'][None, :],
               params['fan2_W'].T, params['fan2_b'][None, :],
               params['node_inits'],
               params['finit_W'][:, :_NDIM].T,
               params['finit_W'][:, _NDIM:_NDIM + _GDIM].T,
               params['finit_W'][:, _NDIM + _GDIM:].T,
               params['finit_b'][None, :],
               params['finit2_W'].T, params['finit2_b'][None, :],
               fs1_W[:, _NDIM:_NDIM + _GDIM].T,
               fs1_W[:, _NDIM + _GDIM:_NDIM + 2 * _GDIM].T,
               fs1_W[:, _NDIM + 2 * _GDIM:].T,
               params['fs1_b'][None, :],
               params['fs2_W'].T, params['fs2_b'][None, :]]
    loss2d, h_v = pl.pallas_call(
        _tail_body,
        out_shape=[jax.ShapeDtypeStruct((_B, 1), f32),
                   jax.ShapeDtypeStruct((_B, _NDIM), f32)],
    )(*tail_in)

    h_out = jnp.concatenate([hn.reshape(_B, _IDX, _NDIM), h_v[:, None, :]],
                            axis=1)
    return (h_out, loss2d[:, 0])


# speedup vs baseline: 3.6592x; 1.0308x over previous
import functools

import jax
import jax.numpy as jnp
from jax import lax
from jax.experimental import pallas as pl
from jax.experimental.pallas import tpu as pltpu
from jax.experimental.pallas import tpu_sc as plsc

_NDIM = 128
_GDIM = 128
_HID = 64
_B = 100
_IDX = 100
_N = _B * _IDX
_NA = 8
_ALPHA = 0.5

_NPAD = 10112
_RPT = 632
_K = 128
_EPT = 20480
_EPAD = 16 * _EPT
_E = 320000
_CHUNKS = _EPT // _K
_G = 2
_GROUPS = _CHUNKS // _G

_ROWS_BLK = 1000
_GRID = _N // _ROWS_BLK


def _dot(a, b):
    return jnp.dot(a, b, preferred_element_type=jnp.float32)


def _make_sc_scatter(do_deg):
    mesh = plsc.VectorSubcoreMesh(core_axis_name="c", subcore_axis_name="s")
    out_type = [jax.ShapeDtypeStruct((2 * _NPAD, _NDIM), jnp.float32)]
    scratch = [
        pltpu.VMEM((_G, 2, _K), jnp.int32),
        pltpu.VMEM((_G, _K, _NDIM), jnp.float32),
        pltpu.VMEM_SHARED((_NPAD, _NDIM), jnp.float32),
        pltpu.SemaphoreType.DMA,
        pltpu.SemaphoreType.DMA,
    ]
    if do_deg:
        out_type.append(jax.ShapeDtypeStruct((2 * _NPAD, _NDIM), jnp.float32))

    def body(*refs):
        if do_deg:
            (tab, idx3, z128, ones128, s_out, deg_out,
             idxb, rows, acc, sem_g, sem_s) = refs
        else:
            (tab, idx3, z128, s_out,
             idxb, rows, acc, sem_g, sem_s) = refs
        cc = lax.axis_index("c")
        ss = lax.axis_index("s")
        rbase = ss * _RPT
        obase = cc * _NPAD + rbase
        irow0 = cc * (_EPAD // _K) + ss * _CHUNKS

        def drain_scatters():
            for u in range(_G):
                pltpu.make_async_copy(z128.at[pl.ds(0, _K)], rows.at[u],
                                      sem_s).wait()

        pltpu.sync_copy(z128, acc.at[pl.ds(rbase, _RPT)])
        if do_deg:
            pltpu.sync_copy(ones128, rows.at[0])
        plsc.subcore_barrier()

        if do_deg:
            def dgroup(g, carry):
                pltpu.sync_copy(idx3.at[pl.ds(irow0 + g * _G, _G)], idxb)
                ds_ = [pltpu.async_copy(rows.at[0], acc.at[idxb.at[u, 1]],
                                        sem_s, add=True) for u in range(_G)]
                for d in ds_:
                    d.wait()
                return carry
            lax.fori_loop(0, _GROUPS, dgroup, 0)
            plsc.subcore_barrier()
            pltpu.sync_copy(acc.at[pl.ds(rbase, _RPT)],
                            deg_out.at[pl.ds(obase, _RPT)])
            pltpu.sync_copy(z128, acc.at[pl.ds(rbase, _RPT)])
            plsc.subcore_barrier()

        def group(g, carry):
            @pl.when(g > 0)
            def _():
                drain_scatters()
            pltpu.sync_copy(idx3.at[pl.ds(irow0 + g * _G, _G)], idxb)
            dg = [pltpu.async_copy(tab.at[idxb.at[u, 0]], rows.at[u], sem_g)
                  for u in range(_G)]
            for d in dg:
                d.wait()
            for u in range(_G):
                pltpu.async_copy(rows.at[u], acc.at[idxb.at[u, 1]], sem_s,
                                 add=True)
            return carry

        lax.fori_loop(0, _GROUPS, group, 0)
        drain_scatters()
        plsc.subcore_barrier()
        pltpu.sync_copy(acc.at[pl.ds(rbase, _RPT)],
                        s_out.at[pl.ds(obase, _RPT)])

    return pl.kernel(body, mesh=mesh, out_type=out_type, scratch_types=scratch)


def _sc_layer1(tab, idx3, z128, ones128):
    res = _make_sc_scatter(True)(tab, idx3, z128, ones128)
    return res[0], res[1]


def _sc_layer2(tab, idx3, z128):
    res = _make_sc_scatter(False)(tab, idx3, z128)
    return res[0] if isinstance(res, (list, tuple)) else res


def _full_spec(arr):
    return pl.BlockSpec(arr.shape, lambda i: tuple(0 for _ in arr.shape))


def _row_spec(ncols):
    return pl.BlockSpec((_ROWS_BLK, ncols), lambda i: (i, 0))


def _gru(x, h, p):
    (wxr, whr, bir, bhr, wxz, whz, biz, bhz, wxn, whn, bin_, bhn) = p
    r = jax.nn.sigmoid(_dot(x, wxr) + bir + _dot(h, whr) + bhr)
    z = jax.nn.sigmoid(_dot(x, wxz) + biz + _dot(h, whz) + bhz)
    n = jnp.tanh(_dot(x, wxn) + bin_ + r * (_dot(h, whn) + bhn))
    return (1.0 - z) * n + z * h


def _pre_body(hf, hb, wlf, wlb, a_out):
    a_out[0] = _dot(hf[...], wlf[...])
    a_out[1] = _dot(hb[...], wlb[...])


def _mid_body(*refs):
    hf, hb, sf, sb, dgf, dgb = refs[:6]
    fw = refs[6:21]
    bw = refs[21:36]
    hf2o, hb2o, af2o = refs[36:39]

    def side(h_ref, s_ref, dg_ref, pr, h2o, a2o, plane):
        h = h_ref[...]
        deg = dg_ref[...][:, 0:1]
        wrT, mb = pr[0][...], pr[1][...]
        aggr = s_ref[...] + deg * (_dot(h, wrT) + mb)
        gru_p = [r[...] for r in pr[2:14]]
        h2 = _gru(aggr, h, gru_p)
        h2o[...] = h2
        a2o[plane] = _dot(h2, pr[14][...])

    side(hf, sf, dgf, fw, hf2o, af2o, 0)
    side(hb, sb, dgb, bw, hb2o, af2o, 1)


def _post_body(*refs):
    hf, hb, sf, sb, dgf, dgb = refs[:6]
    fw = refs[6:20]
    bw = refs[20:34]
    (fm1, fb1, gm1, gb1, fm2, fb2, gm2, gb2, waT) = refs[34:43]
    hno, u1o, u2o, hao = refs[43:47]

    def side(h_ref, s_ref, dg_ref, pr):
        h = h_ref[...]
        deg = dg_ref[...][:, 0:1]
        wrT, mb = pr[0][...], pr[1][...]
        aggr = s_ref[...] + deg * (_dot(h, wrT) + mb)
        gru_p = [r[...] for r in pr[2:14]]
        return _gru(aggr, h, gru_p)

    hf3 = side(hf, sf, dgf, fw)
    hb3 = side(hb, sb, dgb, bw)
    hn = jnp.concatenate([hf3, hb3], axis=1)
    hno[...] = hn
    u1o[...] = (_dot(hn, fm1[...]) + fb1[...]) * jax.nn.sigmoid(
        _dot(hn, gm1[...]) + gb1[...])
    u2o[...] = (_dot(hn, fm2[...]) + fb2[...]) * jax.nn.sigmoid(
        _dot(hn, gm2[...]) + gb2[...])
    hao[...] = _dot(hn, waT[...])


def _tail_body(u1, u2, ha, rm, ttile, trep, c, onehot, edges_flat,
               fanWgT, fanWcT, fanb, fan2T, fan2b, ninits,
               fin_eT, fin_gT, fin_cT, finb, fin2T, fin2b,
               wbT, wcT, wdT, fs1b, fs2T, fs2b,
               loss_o, hv_o):
    R = rm[...]
    hg = _dot(R, u1[...])
    hgi = _dot(R, u2[...])
    cv = c[...]
    s = _dot(hg, fanWgT[...]) + _dot(cv, fanWcT[...]) + fanb[...]
    ns = _dot(jax.nn.relu(s), fan2T[...]) + fan2b[...]
    m = jnp.max(ns, axis=1, keepdims=True)
    logp = ns - m - jnp.log(jnp.sum(jnp.exp(ns - m), axis=1, keepdims=True))
    oh = onehot[...]
    node_loss = -jnp.sum(logp * oh, axis=1, keepdims=True)
    e = _dot(oh, ninits[...])
    t = jax.nn.relu(_dot(e, fin_eT[...]) + _dot(hgi, fin_gT[...]) +
                    _dot(cv, fin_cT[...]) + finb[...])
    hv = _dot(t, fin2T[...]) + fin2b[...]
    hv_o[...] = hv
    P = _dot(hg, wcT[...]) + _dot(cv, wdT[...]) + fs1b[...]
    Q = _dot(hv, wbT[...])
    s2 = ha[...] + _dot(ttile[...], P) + _dot(trep[...], Q)
    es = _dot(jax.nn.relu(s2), fs2T[...]) + fs2b[...]
    ev = edges_flat[...]
    bce = (jnp.maximum(es, 0.0) - es * ev +
           jnp.log(1.0 + jnp.exp(-jnp.abs(es))))
    edge_loss = _dot(R, bce) * (1.0 / _IDX)
    loss_o[...] = 2.0 * ((1.0 - _ALPHA) * node_loss + _ALPHA * edge_loss)


def _prep_layer(p):
    wih, whh = p['Wih'], p['Whh']
    bih, bhh = p['bih'], p['bhh']
    return [
        p['msg_W'][:, _HID:].T,
        p['msg_b'][None, :],
        wih[0:_HID].T, whh[0:_HID].T,
        bih[None, 0:_HID], bhh[None, 0:_HID],
        wih[_HID:2 * _HID].T, whh[_HID:2 * _HID].T,
        bih[None, _HID:2 * _HID], bhh[None, _HID:2 * _HID],
        wih[2 * _HID:].T, whh[2 * _HID:].T,
        bih[None, 2 * _HID:], bhh[None, 2 * _HID:],
    ]


def kernel(h, c, edge_index, node_atts, edges, params):
    f32 = jnp.float32
    h_flat = h.reshape(_N, _NDIM)
    h_f = h_flat[:, :_HID]
    h_b = h_flat[:, _HID:]

    pad_src = jnp.zeros((_EPAD - _E,), jnp.int32)
    pad_dst = jnp.full((_EPAD - _E,), _N, jnp.int32)
    srcf = jnp.concatenate([edge_index[0], pad_src])
    dstf = jnp.concatenate([edge_index[1], pad_dst])
    srcb = jnp.concatenate([edge_index[1], pad_src]) + _N
    dstb = jnp.concatenate([edge_index[0], pad_dst])
    src_all = jnp.concatenate([srcf, srcb]).reshape(2 * _EPAD // _K, _K)
    dst_all = jnp.concatenate([dstf, dstb]).reshape(2 * _EPAD // _K, _K)
    idx3 = jnp.stack([src_all, dst_all], axis=1)

    z128 = jnp.zeros((_RPT, _NDIM), f32)
    ones128 = jnp.ones((_K, _NDIM), f32)

    fl1, fl2 = params['fwd_layers']
    bl1, bl2 = params['bwd_layers']
    wl_f1 = fl1['msg_W'][:, :_HID].T
    wl_b1 = bl1['msg_W'][:, :_HID].T
    wl_f2 = fl2['msg_W'][:, :_HID].T
    wl_b2 = bl2['msg_W'][:, :_HID].T

    a1 = pl.pallas_call(
        _pre_body,
        grid=(_GRID,),
        in_specs=[_row_spec(_HID), _row_spec(_HID),
                  _full_spec(wl_f1), _full_spec(wl_b1)],
        out_specs=pl.BlockSpec((2, _ROWS_BLK, _NDIM), lambda i: (0, i, 0)),
        out_shape=jax.ShapeDtypeStruct((2, _N, _NDIM), f32),
    )(h_f, h_b, wl_f1, wl_b1)

    s1, deg = _sc_layer1(a1.reshape(2 * _N, _NDIM), idx3, z128, ones128)
    s_f1, s_b1 = s1[:_N], s1[_NPAD:_NPAD + _N]
    deg_f, deg_b = deg[:_N], deg[_NPAD:_NPAD + _N]

    mid_params = ([h_f, h_b, s_f1, s_b1, deg_f, deg_b]
                  + _prep_layer(fl1) + [wl_f2]
                  + _prep_layer(bl1) + [wl_b2])
    mid_specs = ([_row_spec(_HID), _row_spec(_HID),
                  _row_spec(_NDIM), _row_spec(_NDIM),
                  _row_spec(_NDIM), _row_spec(_NDIM)]
                 + [_full_spec(a) for a in mid_params[6:]])
    h_f2, h_b2, a2 = pl.pallas_call(
        _mid_body,
        grid=(_GRID,),
        in_specs=mid_specs,
        out_specs=[_row_spec(_HID), _row_spec(_HID),
                   pl.BlockSpec((2, _ROWS_BLK, _NDIM), lambda i: (0, i, 0))],
        out_shape=[jax.ShapeDtypeStruct((_N, _HID), f32),
                   jax.ShapeDtypeStruct((_N, _HID), f32),
                   jax.ShapeDtypeStruct((2, _N, _NDIM), f32)],
    )(*mid_params)

    s2 = _sc_layer2(a2.reshape(2 * _N, _NDIM), idx3, z128)
    s_f2, s_b2 = s2[:_N], s2[_NPAD:_NPAD + _N]

    ge, gei = params['graph_emb'], params['graph_emb_init']
    wa_T = params['fs1_W'][:, :_NDIM].T
    post_params = ([h_f2, h_b2, s_f2, s_b2, deg_f, deg_b]
                   + _prep_layer(fl2) + _prep_layer(bl2)
                   + [ge['fm_W'].T, ge['fm_b'][None, :],
                      ge['gm_W'].T, ge['gm_b'][None, :],
                      gei['fm_W'].T, gei['fm_b'][None, :],
                      gei['gm_W'].T, gei['gm_b'][None, :],
                      wa_T])
    post_specs = ([_row_spec(_HID), _row_spec(_HID),
                   _row_spec(_NDIM), _row_spec(_NDIM),
                   _row_spec(_NDIM), _row_spec(_NDIM)]
                  + [_full_spec(a) for a in post_params[6:]])
    hn, u1, u2, ha = pl.pallas_call(
        _post_body,
        grid=(_GRID,),
        in_specs=post_specs,
        out_specs=[_row_spec(_NDIM), _row_spec(_GDIM),
                   _row_spec(_GDIM), _row_spec(2 * _GDIM)],
        out_shape=[jax.ShapeDtypeStruct((_N, _NDIM), f32),
                   jax.ShapeDtypeStruct((_N, _GDIM), f32),
                   jax.ShapeDtypeStruct((_N, _GDIM), f32),
                   jax.ShapeDtypeStruct((_N, 2 * _GDIM), f32)],
    )(*post_params)

    eye = jnp.eye(_IDX, dtype=f32)
    t_tile = jnp.tile(eye, (_B, 1))
    t_rep = jnp.repeat(eye, _IDX, axis=0)
    rm = t_rep.T
    onehot = jax.nn.one_hot(node_atts, _NA, dtype=f32)
    edges_flat = edges.reshape(_N, 1)
    fan_W, fs1_W = params['fan_W'], params['fs1_W']
    tail_in = [u1, u2, ha, rm, t_tile, t_rep, c, onehot, edges_flat,
               fan_W[:, :_GDIM].T, fan_W[:, _GDIM:].T,
               params['fan_b'][None, :],
               params['fan2_W'].T, params['fan2_b'][None, :],
               params['node_inits'],
               params['finit_W'][:, :_NDIM].T,
               params['finit_W'][:, _NDIM:_NDIM + _GDIM].T,
               params['finit_W'][:, _NDIM + _GDIM:].T,
               params['finit_b'][None, :],
               params['finit2_W'].T, params['finit2_b'][None, :],
               fs1_W[:, _NDIM:_NDIM + _GDIM].T,
               fs1_W[:, _NDIM + _GDIM:_NDIM + 2 * _GDIM].T,
               fs1_W[:, _NDIM + 2 * _GDIM:].T,
               params['fs1_b'][None, :],
               params['fs2_W'].T, params['fs2_b'][None, :]]
    loss2d, h_v = pl.pallas_call(
        _tail_body,
        out_shape=[jax.ShapeDtypeStruct((_B, 1), f32),
                   jax.ShapeDtypeStruct((_B, _NDIM), f32)],
    )(*tail_in)

    h_out = jnp.concatenate([hn.reshape(_B, _IDX, _NDIM), h_v[:, None, :]],
                            axis=1)
    return (h_out, loss2d[:, 0])


# speedup vs baseline: 4.6280x; 1.2647x over previous
import functools

import jax
import jax.numpy as jnp
from jax import lax
from jax.experimental import pallas as pl
from jax.experimental.pallas import tpu as pltpu
from jax.experimental.pallas import tpu_sc as plsc

_NDIM = 128
_GDIM = 128
_HID = 64
_B = 100
_IDX = 100
_N = _B * _IDX
_NA = 8
_ALPHA = 0.5

_NPAD = 10112
_RPT = 632
_K = 128
_EPT = 20352
_EPAD = 16 * _EPT
_E = 320000
_CHUNKS = _EPT // _K
_G = 3
_GROUPS = _CHUNKS // _G

_ROWS_BLK = 1000
_GRID = _N // _ROWS_BLK


def _dot(a, b):
    return jnp.dot(a, b, preferred_element_type=jnp.float32)


def _make_sc_scatter(do_deg):
    mesh = plsc.VectorSubcoreMesh(core_axis_name="c", subcore_axis_name="s")
    out_type = [jax.ShapeDtypeStruct((2 * _NPAD, _NDIM), jnp.float32)]
    scratch = [
        pltpu.VMEM((_G, 2, _K), jnp.int32),
        pltpu.VMEM((_G, _K, _NDIM), jnp.float32),
        pltpu.VMEM_SHARED((_NPAD, _NDIM), jnp.float32),
        pltpu.SemaphoreType.DMA,
        pltpu.SemaphoreType.DMA,
    ]
    if do_deg:
        out_type.append(jax.ShapeDtypeStruct((2 * _NPAD, _NDIM), jnp.float32))

    def body(*refs):
        if do_deg:
            (tab, idx3, z128, ones128, s_out, deg_out,
             idxb, rows, acc, sem_g, sem_s) = refs
        else:
            (tab, idx3, z128, s_out,
             idxb, rows, acc, sem_g, sem_s) = refs
        cc = lax.axis_index("c")
        ss = lax.axis_index("s")
        rbase = ss * _RPT
        obase = cc * _NPAD + rbase
        irow0 = cc * (_EPAD // _K) + ss * _CHUNKS

        def drain_scatters():
            for u in range(_G):
                pltpu.make_async_copy(z128.at[pl.ds(0, _K)], rows.at[u],
                                      sem_s).wait()

        pltpu.sync_copy(z128, acc.at[pl.ds(rbase, _RPT)])
        if do_deg:
            pltpu.sync_copy(ones128, rows.at[0])
        plsc.subcore_barrier()

        if do_deg:
            def dgroup(g, carry):
                pltpu.sync_copy(idx3.at[pl.ds(irow0 + g * _G, _G)], idxb)
                ds_ = [pltpu.async_copy(rows.at[0], acc.at[idxb.at[u, 1]],
                                        sem_s, add=True) for u in range(_G)]
                for d in ds_:
                    d.wait()
                return carry
            lax.fori_loop(0, _GROUPS, dgroup, 0)
            plsc.subcore_barrier()
            pltpu.sync_copy(acc.at[pl.ds(rbase, _RPT)],
                            deg_out.at[pl.ds(obase, _RPT)])
            pltpu.sync_copy(z128, acc.at[pl.ds(rbase, _RPT)])
            plsc.subcore_barrier()

        def group(g, carry):
            @pl.when(g > 0)
            def _():
                drain_scatters()
            pltpu.sync_copy(idx3.at[pl.ds(irow0 + g * _G, _G)], idxb)
            dg = [pltpu.async_copy(tab.at[idxb.at[u, 0]], rows.at[u], sem_g)
                  for u in range(_G)]
            for d in dg:
                d.wait()
            for u in range(_G):
                pltpu.async_copy(rows.at[u], acc.at[idxb.at[u, 1]], sem_s,
                                 add=True)
            return carry

        lax.fori_loop(0, _GROUPS, group, 0)
        drain_scatters()
        plsc.subcore_barrier()
        pltpu.sync_copy(acc.at[pl.ds(rbase, _RPT)],
                        s_out.at[pl.ds(obase, _RPT)])

    return pl.kernel(body, mesh=mesh, out_type=out_type, scratch_types=scratch)


def _sc_layer1(tab, idx3, z128, ones128):
    res = _make_sc_scatter(True)(tab, idx3, z128, ones128)
    return res[0], res[1]


def _sc_layer2(tab, idx3, z128):
    res = _make_sc_scatter(False)(tab, idx3, z128)
    return res[0] if isinstance(res, (list, tuple)) else res


def _full_spec(arr):
    return pl.BlockSpec(arr.shape, lambda i: tuple(0 for _ in arr.shape))


def _row_spec(ncols):
    return pl.BlockSpec((_ROWS_BLK, ncols), lambda i: (i, 0))


def _gru(x, h, p):
    (wxr, whr, bir, bhr, wxz, whz, biz, bhz, wxn, whn, bin_, bhn) = p
    r = jax.nn.sigmoid(_dot(x, wxr) + bir + _dot(h, whr) + bhr)
    z = jax.nn.sigmoid(_dot(x, wxz) + biz + _dot(h, whz) + bhz)
    n = jnp.tanh(_dot(x, wxn) + bin_ + r * (_dot(h, whn) + bhn))
    return (1.0 - z) * n + z * h


def _pre_body(hf, hb, wlf, wlb, a_out):
    a_out[0] = _dot(hf[...], wlf[...])
    a_out[1] = _dot(hb[...], wlb[...])


def _mid_body(*refs):
    hf, hb, sf, sb, dgf, dgb = refs[:6]
    fw = refs[6:21]
    bw = refs[21:36]
    hf2o, hb2o, af2o = refs[36:39]

    def side(h_ref, s_ref, dg_ref, pr, h2o, a2o, plane):
        h = h_ref[...]
        deg = dg_ref[...][:, 0:1]
        wrT, mb = pr[0][...], pr[1][...]
        aggr = s_ref[...] + deg * (_dot(h, wrT) + mb)
        gru_p = [r[...] for r in pr[2:14]]
        h2 = _gru(aggr, h, gru_p)
        h2o[...] = h2
        a2o[plane] = _dot(h2, pr[14][...])

    side(hf, sf, dgf, fw, hf2o, af2o, 0)
    side(hb, sb, dgb, bw, hb2o, af2o, 1)


def _post_body(*refs):
    hf, hb, sf, sb, dgf, dgb = refs[:6]
    fw = refs[6:20]
    bw = refs[20:34]
    (fm1, fb1, gm1, gb1, fm2, fb2, gm2, gb2, waT) = refs[34:43]
    hno, u1o, u2o, hao = refs[43:47]

    def side(h_ref, s_ref, dg_ref, pr):
        h = h_ref[...]
        deg = dg_ref[...][:, 0:1]
        wrT, mb = pr[0][...], pr[1][...]
        aggr = s_ref[...] + deg * (_dot(h, wrT) + mb)
        gru_p = [r[...] for r in pr[2:14]]
        return _gru(aggr, h, gru_p)

    hf3 = side(hf, sf, dgf, fw)
    hb3 = side(hb, sb, dgb, bw)
    hn = jnp.concatenate([hf3, hb3], axis=1)
    hno[...] = hn
    u1o[...] = (_dot(hn, fm1[...]) + fb1[...]) * jax.nn.sigmoid(
        _dot(hn, gm1[...]) + gb1[...])
    u2o[...] = (_dot(hn, fm2[...]) + fb2[...]) * jax.nn.sigmoid(
        _dot(hn, gm2[...]) + gb2[...])
    hao[...] = _dot(hn, waT[...])


def _tail_body(u1, u2, ha, rm, ttile, trep, c, onehot, edges_flat,
               fanWgT, fanWcT, fanb, fan2T, fan2b, ninits,
               fin_eT, fin_gT, fin_cT, finb, fin2T, fin2b,
               wbT, wcT, wdT, fs1b, fs2T, fs2b,
               loss_o, hv_o):
    R = rm[...]
    hg = _dot(R, u1[...])
    hgi = _dot(R, u2[...])
    cv = c[...]
    s = _dot(hg, fanWgT[...]) + _dot(cv, fanWcT[...]) + fanb[...]
    ns = _dot(jax.nn.relu(s), fan2T[...]) + fan2b[...]
    m = jnp.max(ns, axis=1, keepdims=True)
    logp = ns - m - jnp.log(jnp.sum(jnp.exp(ns - m), axis=1, keepdims=True))
    oh = onehot[...]
    node_loss = -jnp.sum(logp * oh, axis=1, keepdims=True)
    e = _dot(oh, ninits[...])
    t = jax.nn.relu(_dot(e, fin_eT[...]) + _dot(hgi, fin_gT[...]) +
                    _dot(cv, fin_cT[...]) + finb[...])
    hv = _dot(t, fin2T[...]) + fin2b[...]
    hv_o[...] = hv
    P = _dot(hg, wcT[...]) + _dot(cv, wdT[...]) + fs1b[...]
    Q = _dot(hv, wbT[...])
    s2 = ha[...] + _dot(ttile[...], P) + _dot(trep[...], Q)
    es = _dot(jax.nn.relu(s2), fs2T[...]) + fs2b[...]
    ev = edges_flat[...]
    bce = (jnp.maximum(es, 0.0) - es * ev +
           jnp.log(1.0 + jnp.exp(-jnp.abs(es))))
    edge_loss = _dot(R, bce) * (1.0 / _IDX)
    loss_o[...] = 2.0 * ((1.0 - _ALPHA) * node_loss + _ALPHA * edge_loss)


def _prep_layer(p):
    wih, whh = p['Wih'], p['Whh']
    bih, bhh = p['bih'], p['bhh']
    return [
        p['msg_W'][:, _HID:].T,
        p['msg_b'][None, :],
        wih[0:_HID].T, whh[0:_HID].T,
        bih[None, 0:_HID], bhh[None, 0:_HID],
        wih[_HID:2 * _HID].T, whh[_HID:2 * _HID].T,
        bih[None, _HID:2 * _HID], bhh[None, _HID:2 * _HID],
        wih[2 * _HID:].T, whh[2 * _HID:].T,
        bih[None, 2 * _HID:], bhh[None, 2 * _HID:],
    ]


def kernel(h, c, edge_index, node_atts, edges, params):
    f32 = jnp.float32
    h_flat = h.reshape(_N, _NDIM)
    h_f = h_flat[:, :_HID]
    h_b = h_flat[:, _HID:]

    pad_src = jnp.zeros((_EPAD - _E,), jnp.int32)
    pad_dst = jnp.full((_EPAD - _E,), _N, jnp.int32)
    srcf = jnp.concatenate([edge_index[0], pad_src])
    dstf = jnp.concatenate([edge_index[1], pad_dst])
    srcb = jnp.concatenate([edge_index[1], pad_src]) + _N
    dstb = jnp.concatenate([edge_index[0], pad_dst])
    src_all = jnp.concatenate([srcf, srcb]).reshape(2 * _EPAD // _K, _K)
    dst_all = jnp.concatenate([dstf, dstb]).reshape(2 * _EPAD // _K, _K)
    idx3 = jnp.stack([src_all, dst_all], axis=1)

    z128 = jnp.zeros((_RPT, _NDIM), f32)
    ones128 = jnp.ones((_K, _NDIM), f32)

    fl1, fl2 = params['fwd_layers']
    bl1, bl2 = params['bwd_layers']
    wl_f1 = fl1['msg_W'][:, :_HID].T
    wl_b1 = bl1['msg_W'][:, :_HID].T
    wl_f2 = fl2['msg_W'][:, :_HID].T
    wl_b2 = bl2['msg_W'][:, :_HID].T

    a1 = pl.pallas_call(
        _pre_body,
        grid=(_GRID,),
        in_specs=[_row_spec(_HID), _row_spec(_HID),
                  _full_spec(wl_f1), _full_spec(wl_b1)],
        out_specs=pl.BlockSpec((2, _ROWS_BLK, _NDIM), lambda i: (0, i, 0)),
        out_shape=jax.ShapeDtypeStruct((2, _N, _NDIM), f32),
    )(h_f, h_b, wl_f1, wl_b1)

    s1, deg = _sc_layer1(a1.reshape(2 * _N, _NDIM), idx3, z128, ones128)
    s_f1, s_b1 = s1[:_N], s1[_NPAD:_NPAD + _N]
    deg_f, deg_b = deg[:_N], deg[_NPAD:_NPAD + _N]

    mid_params = ([h_f, h_b, s_f1, s_b1, deg_f, deg_b]
                  + _prep_layer(fl1) + [wl_f2]
                  + _prep_layer(bl1) + [wl_b2])
    mid_specs = ([_row_spec(_HID), _row_spec(_HID),
                  _row_spec(_NDIM), _row_spec(_NDIM),
                  _row_spec(_NDIM), _row_spec(_NDIM)]
                 + [_full_spec(a) for a in mid_params[6:]])
    h_f2, h_b2, a2 = pl.pallas_call(
        _mid_body,
        grid=(_GRID,),
        in_specs=mid_specs,
        out_specs=[_row_spec(_HID), _row_spec(_HID),
                   pl.BlockSpec((2, _ROWS_BLK, _NDIM), lambda i: (0, i, 0))],
        out_shape=[jax.ShapeDtypeStruct((_N, _HID), f32),
                   jax.ShapeDtypeStruct((_N, _HID), f32),
                   jax.ShapeDtypeStruct((2, _N, _NDIM), f32)],
    )(*mid_params)

    s2 = _sc_layer2(a2.reshape(2 * _N, _NDIM), idx3, z128)
    s_f2, s_b2 = s2[:_N], s2[_NPAD:_NPAD + _N]

    ge, gei = params['graph_emb'], params['graph_emb_init']
    wa_T = params['fs1_W'][:, :_NDIM].T
    post_params = ([h_f2, h_b2, s_f2, s_b2, deg_f, deg_b]
                   + _prep_layer(fl2) + _prep_layer(bl2)
                   + [ge['fm_W'].T, ge['fm_b'][None, :],
                      ge['gm_W'].T, ge['gm_b'][None, :],
                      gei['fm_W'].T, gei['fm_b'][None, :],
                      gei['gm_W'].T, gei['gm_b'][None, :],
                      wa_T])
    post_specs = ([_row_spec(_HID), _row_spec(_HID),
                   _row_spec(_NDIM), _row_spec(_NDIM),
                   _row_spec(_NDIM), _row_spec(_NDIM)]
                  + [_full_spec(a) for a in post_params[6:]])
    hn, u1, u2, ha = pl.pallas_call(
        _post_body,
        grid=(_GRID,),
        in_specs=post_specs,
        out_specs=[_row_spec(_NDIM), _row_spec(_GDIM),
                   _row_spec(_GDIM), _row_spec(2 * _GDIM)],
        out_shape=[jax.ShapeDtypeStruct((_N, _NDIM), f32),
                   jax.ShapeDtypeStruct((_N, _GDIM), f32),
                   jax.ShapeDtypeStruct((_N, _GDIM), f32),
                   jax.ShapeDtypeStruct((_N, 2 * _GDIM), f32)],
    )(*post_params)

    eye = jnp.eye(_IDX, dtype=f32)
    t_tile = jnp.tile(eye, (_B, 1))
    t_rep = jnp.repeat(eye, _IDX, axis=0)
    rm = t_rep.T
    onehot = jax.nn.one_hot(node_atts, _NA, dtype=f32)
    edges_flat = edges.reshape(_N, 1)
    fan_W, fs1_W = params['fan_W'], params['fs1_W']
    tail_in = [u1, u2, ha, rm, t_tile, t_rep, c, onehot, edges_flat,
               fan_W[:, :_GDIM].T, fan_W[:, _GDIM:].T,
               params['fan_b'][None, :],
               params['fan2_W'].T, params['fan2_b'][None, :],
               params['node_inits'],
               params['finit_W'][:, :_NDIM].T,
               params['finit_W'][:, _NDIM:_NDIM + _GDIM].T,
               params['finit_W'][:, _NDIM + _GDIM:].T,
               params['finit_b'][None, :],
               params['finit2_W'].T, params['finit2_b'][None, :],
               fs1_W[:, _NDIM:_NDIM + _GDIM].T,
               fs1_W[:, _NDIM + _GDIM:_NDIM + 2 * _GDIM].T,
               fs1_W[:, _NDIM + 2 * _GDIM:].T,
               params['fs1_b'][None, :],
               params['fs2_W'].T, params['fs2_b'][None, :]]
    loss2d, h_v = pl.pallas_call(
        _tail_body,
        out_shape=[jax.ShapeDtypeStruct((_B, 1), f32),
                   jax.ShapeDtypeStruct((_B, _NDIM), f32)],
    )(*tail_in)

    h_out = jnp.concatenate([hn.reshape(_B, _IDX, _NDIM), h_v[:, None, :]],
                            axis=1)
    return (h_out, loss2d[:, 0])


# speedup vs baseline: 4.6344x; 1.0014x over previous
import functools

import jax
import jax.numpy as jnp
from jax import lax
from jax.experimental import pallas as pl
from jax.experimental.pallas import tpu as pltpu
from jax.experimental.pallas import tpu_sc as plsc

_NDIM = 128
_GDIM = 128
_HID = 64
_B = 100
_IDX = 100
_N = _B * _IDX
_NA = 8
_ALPHA = 0.5

_NPAD = 10112
_RPT = 632
_K = 128
_EPT = 20352
_EPAD = 16 * _EPT
_E = 320000
_CHUNKS = _EPT // _K
_G = 3
_GROUPS = _CHUNKS // _G

_ROWS_BLK = 1000
_GRID = _N // _ROWS_BLK


def _dot(a, b):
    return jnp.dot(a, b, preferred_element_type=jnp.float32)


def _make_sc_scatter(do_deg):
    mesh = plsc.VectorSubcoreMesh(core_axis_name="c", subcore_axis_name="s")
    out_type = [jax.ShapeDtypeStruct((2 * _NPAD, _NDIM), jnp.float32)]
    scratch = [
        pltpu.VMEM((_G, 2, _K), jnp.int32),
        pltpu.VMEM((_G, _K, _NDIM), jnp.float32),
        pltpu.VMEM_SHARED((_NPAD, _NDIM), jnp.float32),
        pltpu.SemaphoreType.DMA,
        pltpu.SemaphoreType.DMA,
    ]
    if do_deg:
        out_type.append(jax.ShapeDtypeStruct((2 * _NPAD, _NDIM), jnp.float32))

    def body(*refs):
        if do_deg:
            (tab, idx3, z128, ones128, s_out, deg_out,
             idxb, rows, acc, sem_g, sem_s) = refs
        else:
            (tab, idx3, z128, s_out,
             idxb, rows, acc, sem_g, sem_s) = refs
        cc = lax.axis_index("c")
        ss = lax.axis_index("s")
        rbase = ss * _RPT
        obase = cc * _NPAD + rbase
        irow0 = cc * (_EPAD // _K) + ss * _CHUNKS

        def drain_scatters():
            for u in range(_G):
                pltpu.make_async_copy(z128.at[pl.ds(0, _K)], rows.at[u],
                                      sem_s).wait()

        pltpu.sync_copy(z128, acc.at[pl.ds(rbase, _RPT)])
        if do_deg:
            pltpu.sync_copy(ones128, rows.at[0])
        plsc.subcore_barrier()

        if do_deg:
            def dgroup(g, carry):
                @pl.when(g > 0)
                def _():
                    drain_scatters()
                pltpu.sync_copy(idx3.at[pl.ds(irow0 + g * _G, _G)], idxb)
                for u in range(_G):
                    pltpu.async_copy(rows.at[0], acc.at[idxb.at[u, 1]],
                                     sem_s, add=True)
                return carry
            lax.fori_loop(0, _GROUPS, dgroup, 0)
            drain_scatters()
            plsc.subcore_barrier()
            pltpu.sync_copy(acc.at[pl.ds(rbase, _RPT)],
                            deg_out.at[pl.ds(obase, _RPT)])
            pltpu.sync_copy(z128, acc.at[pl.ds(rbase, _RPT)])
            plsc.subcore_barrier()

        def group(g, carry):
            @pl.when(g > 0)
            def _():
                drain_scatters()
            pltpu.sync_copy(idx3.at[pl.ds(irow0 + g * _G, _G)], idxb)
            dg = [pltpu.async_copy(tab.at[idxb.at[u, 0]], rows.at[u], sem_g)
                  for u in range(_G)]
            for d in dg:
                d.wait()
            for u in range(_G):
                pltpu.async_copy(rows.at[u], acc.at[idxb.at[u, 1]], sem_s,
                                 add=True)
            return carry

        lax.fori_loop(0, _GROUPS, group, 0)
        drain_scatters()
        plsc.subcore_barrier()
        pltpu.sync_copy(acc.at[pl.ds(rbase, _RPT)],
                        s_out.at[pl.ds(obase, _RPT)])

    return pl.kernel(body, mesh=mesh, out_type=out_type, scratch_types=scratch)


def _sc_layer1(tab, idx3, z128, ones128):
    res = _make_sc_scatter(True)(tab, idx3, z128, ones128)
    return res[0], res[1]


def _sc_layer2(tab, idx3, z128):
    res = _make_sc_scatter(False)(tab, idx3, z128)
    return res[0] if isinstance(res, (list, tuple)) else res


def _full_spec(arr):
    return pl.BlockSpec(arr.shape, lambda i: tuple(0 for _ in arr.shape))


def _row_spec(ncols):
    return pl.BlockSpec((_ROWS_BLK, ncols), lambda i: (i, 0))


def _gru(x, h, p):
    (wxr, whr, bir, bhr, wxz, whz, biz, bhz, wxn, whn, bin_, bhn) = p
    r = jax.nn.sigmoid(_dot(x, wxr) + bir + _dot(h, whr) + bhr)
    z = jax.nn.sigmoid(_dot(x, wxz) + biz + _dot(h, whz) + bhz)
    n = jnp.tanh(_dot(x, wxn) + bin_ + r * (_dot(h, whn) + bhn))
    return (1.0 - z) * n + z * h


def _pre_body(hf, hb, wlf, wlb, a_out):
    a_out[0] = _dot(hf[...], wlf[...])
    a_out[1] = _dot(hb[...], wlb[...])


def _mid_body(*refs):
    hf, hb, sf, sb, dgf, dgb = refs[:6]
    fw = refs[6:21]
    bw = refs[21:36]
    hf2o, hb2o, af2o = refs[36:39]

    def side(h_ref, s_ref, dg_ref, pr, h2o, a2o, plane):
        h = h_ref[...]
        deg = dg_ref[...][:, 0:1]
        wrT, mb = pr[0][...], pr[1][...]
        aggr = s_ref[...] + deg * (_dot(h, wrT) + mb)
        gru_p = [r[...] for r in pr[2:14]]
        h2 = _gru(aggr, h, gru_p)
        h2o[...] = h2
        a2o[plane] = _dot(h2, pr[14][...])

    side(hf, sf, dgf, fw, hf2o, af2o, 0)
    side(hb, sb, dgb, bw, hb2o, af2o, 1)


def _post_body(*refs):
    hf, hb, sf, sb, dgf, dgb = refs[:6]
    fw = refs[6:20]
    bw = refs[20:34]
    (fm1, fb1, gm1, gb1, fm2, fb2, gm2, gb2, waT) = refs[34:43]
    hno, u1o, u2o, hao = refs[43:47]

    def side(h_ref, s_ref, dg_ref, pr):
        h = h_ref[...]
        deg = dg_ref[...][:, 0:1]
        wrT, mb = pr[0][...], pr[1][...]
        aggr = s_ref[...] + deg * (_dot(h, wrT) + mb)
        gru_p = [r[...] for r in pr[2:14]]
        return _gru(aggr, h, gru_p)

    hf3 = side(hf, sf, dgf, fw)
    hb3 = side(hb, sb, dgb, bw)
    hn = jnp.concatenate([hf3, hb3], axis=1)
    hno[...] = hn
    u1o[...] = (_dot(hn, fm1[...]) + fb1[...]) * jax.nn.sigmoid(
        _dot(hn, gm1[...]) + gb1[...])
    u2o[...] = (_dot(hn, fm2[...]) + fb2[...]) * jax.nn.sigmoid(
        _dot(hn, gm2[...]) + gb2[...])
    hao[...] = _dot(hn, waT[...])


def _tail_body(u1, u2, ha, rm, ttile, trep, c, onehot, edges_flat,
               fanWgT, fanWcT, fanb, fan2T, fan2b, ninits,
               fin_eT, fin_gT, fin_cT, finb, fin2T, fin2b,
               wbT, wcT, wdT, fs1b, fs2T, fs2b,
               loss_o, hv_o):
    R = rm[...]
    hg = _dot(R, u1[...])
    hgi = _dot(R, u2[...])
    cv = c[...]
    s = _dot(hg, fanWgT[...]) + _dot(cv, fanWcT[...]) + fanb[...]
    ns = _dot(jax.nn.relu(s), fan2T[...]) + fan2b[...]
    m = jnp.max(ns, axis=1, keepdims=True)
    logp = ns - m - jnp.log(jnp.sum(jnp.exp(ns - m), axis=1, keepdims=True))
    oh = onehot[...]
    node_loss = -jnp.sum(logp * oh, axis=1, keepdims=True)
    e = _dot(oh, ninits[...])
    t = jax.nn.relu(_dot(e, fin_eT[...]) + _dot(hgi, fin_gT[...]) +
                    _dot(cv, fin_cT[...]) + finb[...])
    hv = _dot(t, fin2T[...]) + fin2b[...]
    hv_o[...] = hv
    P = _dot(hg, wcT[...]) + _dot(cv, wdT[...]) + fs1b[...]
    Q = _dot(hv, wbT[...])
    s2 = ha[...] + _dot(ttile[...], P) + _dot(trep[...], Q)
    es = _dot(jax.nn.relu(s2), fs2T[...]) + fs2b[...]
    ev = edges_flat[...]
    bce = (jnp.maximum(es, 0.0) - es * ev +
           jnp.log(1.0 + jnp.exp(-jnp.abs(es))))
    edge_loss = _dot(R, bce) * (1.0 / _IDX)
    loss_o[...] = 2.0 * ((1.0 - _ALPHA) * node_loss + _ALPHA * edge_loss)


def _prep_layer(p):
    wih, whh = p['Wih'], p['Whh']
    bih, bhh = p['bih'], p['bhh']
    return [
        p['msg_W'][:, _HID:].T,
        p['msg_b'][None, :],
        wih[0:_HID].T, whh[0:_HID].T,
        bih[None, 0:_HID], bhh[None, 0:_HID],
        wih[_HID:2 * _HID].T, whh[_HID:2 * _HID].T,
        bih[None, _HID:2 * _HID], bhh[None, _HID:2 * _HID],
        wih[2 * _HID:].T, whh[2 * _HID:].T,
        bih[None, 2 * _HID:], bhh[None, 2 * _HID:],
    ]


def kernel(h, c, edge_index, node_atts, edges, params):
    f32 = jnp.float32
    h_flat = h.reshape(_N, _NDIM)
    h_f = h_flat[:, :_HID]
    h_b = h_flat[:, _HID:]

    pad_src = jnp.zeros((_EPAD - _E,), jnp.int32)
    pad_dst = jnp.full((_EPAD - _E,), _N, jnp.int32)
    srcf = jnp.concatenate([edge_index[0], pad_src])
    dstf = jnp.concatenate([edge_index[1], pad_dst])
    srcb = jnp.concatenate([edge_index[1], pad_src]) + _N
    dstb = jnp.concatenate([edge_index[0], pad_dst])
    src_all = jnp.concatenate([srcf, srcb]).reshape(2 * _EPAD // _K, _K)
    dst_all = jnp.concatenate([dstf, dstb]).reshape(2 * _EPAD // _K, _K)
    idx3 = jnp.stack([src_all, dst_all], axis=1)

    z128 = jnp.zeros((_RPT, _NDIM), f32)
    ones128 = jnp.ones((_K, _NDIM), f32)

    fl1, fl2 = params['fwd_layers']
    bl1, bl2 = params['bwd_layers']
    wl_f1 = fl1['msg_W'][:, :_HID].T
    wl_b1 = bl1['msg_W'][:, :_HID].T
    wl_f2 = fl2['msg_W'][:, :_HID].T
    wl_b2 = bl2['msg_W'][:, :_HID].T

    a1 = pl.pallas_call(
        _pre_body,
        grid=(_GRID,),
        in_specs=[_row_spec(_HID), _row_spec(_HID),
                  _full_spec(wl_f1), _full_spec(wl_b1)],
        out_specs=pl.BlockSpec((2, _ROWS_BLK, _NDIM), lambda i: (0, i, 0)),
        out_shape=jax.ShapeDtypeStruct((2, _N, _NDIM), f32),
    )(h_f, h_b, wl_f1, wl_b1)

    s1, deg = _sc_layer1(a1.reshape(2 * _N, _NDIM), idx3, z128, ones128)
    s_f1, s_b1 = s1[:_N], s1[_NPAD:_NPAD + _N]
    deg_f, deg_b = deg[:_N], deg[_NPAD:_NPAD + _N]

    mid_params = ([h_f, h_b, s_f1, s_b1, deg_f, deg_b]
                  + _prep_layer(fl1) + [wl_f2]
                  + _prep_layer(bl1) + [wl_b2])
    mid_specs = ([_row_spec(_HID), _row_spec(_HID),
                  _row_spec(_NDIM), _row_spec(_NDIM),
                  _row_spec(_NDIM), _row_spec(_NDIM)]
                 + [_full_spec(a) for a in mid_params[6:]])
    h_f2, h_b2, a2 = pl.pallas_call(
        _mid_body,
        grid=(_GRID,),
        in_specs=mid_specs,
        out_specs=[_row_spec(_HID), _row_spec(_HID),
                   pl.BlockSpec((2, _ROWS_BLK, _NDIM), lambda i: (0, i, 0))],
        out_shape=[jax.ShapeDtypeStruct((_N, _HID), f32),
                   jax.ShapeDtypeStruct((_N, _HID), f32),
                   jax.ShapeDtypeStruct((2, _N, _NDIM), f32)],
    )(*mid_params)

    s2 = _sc_layer2(a2.reshape(2 * _N, _NDIM), idx3, z128)
    s_f2, s_b2 = s2[:_N], s2[_NPAD:_NPAD + _N]

    ge, gei = params['graph_emb'], params['graph_emb_init']
    wa_T = params['fs1_W'][:, :_NDIM].T
    post_params = ([h_f2, h_b2, s_f2, s_b2, deg_f, deg_b]
                   + _prep_layer(fl2) + _prep_layer(bl2)
                   + [ge['fm_W'].T, ge['fm_b'][None, :],
                      ge['gm_W'].T, ge['gm_b'][None, :],
                      gei['fm_W'].T, gei['fm_b'][None, :],
                      gei['gm_W'].T, gei['gm_b'][None, :],
                      wa_T])
    post_specs = ([_row_spec(_HID), _row_spec(_HID),
                   _row_spec(_NDIM), _row_spec(_NDIM),
                   _row_spec(_NDIM), _row_spec(_NDIM)]
                  + [_full_spec(a) for a in post_params[6:]])
    hn, u1, u2, ha = pl.pallas_call(
        _post_body,
        grid=(_GRID,),
        in_specs=post_specs,
        out_specs=[_row_spec(_NDIM), _row_spec(_GDIM),
                   _row_spec(_GDIM), _row_spec(2 * _GDIM)],
        out_shape=[jax.ShapeDtypeStruct((_N, _NDIM), f32),
                   jax.ShapeDtypeStruct((_N, _GDIM), f32),
                   jax.ShapeDtypeStruct((_N, _GDIM), f32),
                   jax.ShapeDtypeStruct((_N, 2 * _GDIM), f32)],
    )(*post_params)

    eye = jnp.eye(_IDX, dtype=f32)
    t_tile = jnp.tile(eye, (_B, 1))
    t_rep = jnp.repeat(eye, _IDX, axis=0)
    rm = t_rep.T
    onehot = jax.nn.one_hot(node_atts, _NA, dtype=f32)
    edges_flat = edges.reshape(_N, 1)
    fan_W, fs1_W = params['fan_W'], params['fs1_W']
    tail_in = [u1, u2, ha, rm, t_tile, t_rep, c, onehot, edges_flat,
               fan_W[:, :_GDIM].T, fan_W[:, _GDIM:].T,
               params['fan_b'][None, :],
               params['fan2_W'].T, params['fan2_b'][None, :],
               params['node_inits'],
               params['finit_W'][:, :_NDIM].T,
               params['finit_W'][:, _NDIM:_NDIM + _GDIM].T,
               params['finit_W'][:, _NDIM + _GDIM:].T,
               params['finit_b'][None, :],
               params['finit2_W'].T, params['finit2_b'][None, :],
               fs1_W[:, _NDIM:_NDIM + _GDIM].T,
               fs1_W[:, _NDIM + _GDIM:_NDIM + 2 * _GDIM].T,
               fs1_W[:, _NDIM + 2 * _GDIM:].T,
               params['fs1_b'][None, :],
               params['fs2_W'].T, params['fs2_b'][None, :]]
    loss2d, h_v = pl.pallas_call(
        _tail_body,
        out_shape=[jax.ShapeDtypeStruct((_B, 1), f32),
                   jax.ShapeDtypeStruct((_B, _NDIM), f32)],
    )(*tail_in)

    h_out = jnp.concatenate([hn.reshape(_B, _IDX, _NDIM), h_v[:, None, :]],
                            axis=1)
    return (h_out, loss2d[:, 0])
